# baseline (speedup 1.0000x reference)
import functools

import numpy as np
import jax
import jax.numpy as jnp
from jax import lax
from jax.experimental import pallas as pl
from jax.experimental.pallas import tpu as pltpu
from jax.experimental.pallas import tpu_sc as plsc

_BN_EPS = 1e-5
_NC, _NS = 2, 16
_NW = _NC * _NS


def _fps_kernel(rows_ref, x_ref, y_ref, z_ref, o_ref, d_ref, *, npoint, n):
    nl = n // 8
    x = x_ref[...]
    y = y_ref[...]
    z = z_ref[...]
    i0 = lax.broadcasted_iota(jnp.int32, (16, nl), 0)
    nidx = (i0 % 8) * nl + lax.broadcasted_iota(jnp.int32, (16, nl), 1)
    half = lax.broadcasted_iota(jnp.int32, (16, 1), 0) < 8
    d_ref[...] = jnp.full((16, nl), 1e10, dtype=jnp.float32)

    def step(t, carry):
        f0, f1 = carry
        r0 = rows_ref[pl.ds(f0, 1), :]
        r1 = rows_ref[pl.ds(f1 + n, 1), :]
        o_ref[0, pl.ds(t, 1), :] = r0
        o_ref[1, pl.ds(t, 1), :] = r1
        cx = jnp.where(half, jnp.sum(r0[:, 0:1]), jnp.sum(r1[:, 0:1]))
        cy = jnp.where(half, jnp.sum(r0[:, 1:2]), jnp.sum(r1[:, 1:2]))
        cz = jnp.where(half, jnp.sum(r0[:, 2:3]), jnp.sum(r1[:, 2:3]))
        dx = x - cx
        dy = y - cy
        dz = z - cz
        d = dx * dx + dy * dy + dz * dz
        dn = jnp.minimum(d_ref[...], d)
        d_ref[...] = dn
        mx0 = jnp.max(dn[0:8])
        mx1 = jnp.max(dn[8:16])
        nf0 = jnp.min(jnp.where(dn[0:8] == mx0, nidx[0:8], n)).astype(jnp.int32)
        nf1 = jnp.min(jnp.where(dn[8:16] == mx1, nidx[8:16], n)).astype(jnp.int32)
        return nf0, nf1

    lax.fori_loop(0, npoint, step, (jnp.int32(0), jnp.int32(0)))


def _fps(xyz, npoint):
    b, n, _ = xyz.shape
    nl = n // 8
    rows = xyz.reshape(b * n, 3)
    xs = xyz[..., 0].reshape(b * 8, nl)
    ys = xyz[..., 1].reshape(b * 8, nl)
    zs = xyz[..., 2].reshape(b * 8, nl)
    return pl.pallas_call(
        functools.partial(_fps_kernel, npoint=npoint, n=n),
        out_shape=jax.ShapeDtypeStruct((b, npoint, 3), jnp.float32),
        scratch_shapes=[pltpu.VMEM((16, nl), jnp.float32)],
    )(rows, xs, ys, zs)


def _ball_kernel(xr, yr, zr, cxr, cyr, czr, o_ref, sel_ref, cnt_ref,
                 *, r2, ns, n, nch):
    ch = n // nch
    b = pl.program_id(0)
    cx = cxr[0, 0]
    cy = cyr[0, 0]
    cz = czr[0, 0]
    liota = lax.broadcasted_iota(jnp.int32, (8, ch), 1)
    coliota = lax.broadcasted_iota(jnp.int32, (8, ns), 1)
    sel_ref[...] = jnp.full((8, ns), n, jnp.int32)
    cnt_ref[...] = jnp.zeros((8, ns), jnp.int32)

    def chunk_body(c, _):
        @pl.when(jnp.min(cnt_ref[:, 0:1]) < ns)
        def _():
            xs = xr[0, pl.ds(c, 1), :]
            ys = yr[0, pl.ds(c, 1), :]
            zs = zr[0, pl.ds(c, 1), :]
            dx = cx - xs
            dy = cy - ys
            dz = cz - zs
            d2 = dx * dx + dy * dy + dz * dz
            cand0 = jnp.where(d2 <= r2, liota + c * ch, n)

            def slot_body(s, cand):
                v = jnp.min(cand, axis=1, keepdims=True)
                cnt = cnt_ref[:, 0:1]
                place = (v < n) & (cnt < ns)
                sel_ref[...] = jnp.where(place & (coliota == cnt), v,
                                         sel_ref[...])
                cnt_ref[...] = cnt_ref[...] + place.astype(jnp.int32)
                return jnp.where(cand == v, n, cand)

            lax.fori_loop(0, ns, slot_body, cand0)

        return 0

    lax.fori_loop(0, nch, chunk_body, 0)
    idx = sel_ref[...]
    first = idx[:, 0:1]
    idx = jnp.where(idx == n, first, idx)
    idx = jnp.where(idx == n, 0, idx)
    o_ref[0, 0] = idx + b * n


def _ball(xyz, new_xyz, radius, ns):
    b, n, _ = xyz.shape
    nch = 8 if n >= 4096 else 4
    ch = n // nch
    npoint = new_xyz.shape[1]
    xr = xyz[..., 0].reshape(b, nch, ch)
    yr = xyz[..., 1].reshape(b, nch, ch)
    zr = xyz[..., 2].reshape(b, nch, ch)
    cx = new_xyz[..., 0].reshape(b, npoint // 8, 8, 1)
    cy = new_xyz[..., 1].reshape(b, npoint // 8, 8, 1)
    cz = new_xyz[..., 2].reshape(b, npoint // 8, 8, 1)
    out = pl.pallas_call(
        functools.partial(_ball_kernel, r2=radius * radius, ns=ns, n=n,
                          nch=nch),
        grid=(b, npoint // 8),
        in_specs=[pl.BlockSpec((1, nch, ch), lambda i, j: (i, 0, 0))] * 3
        + [pl.BlockSpec((1, 1, 8, 1), lambda i, j: (i, j, 0, 0))] * 3,
        out_specs=pl.BlockSpec((1, 1, 8, ns), lambda i, j: (i, j, 0, 0)),
        out_shape=jax.ShapeDtypeStruct((b, npoint // 8, 8, ns), jnp.int32),
        scratch_shapes=[pltpu.VMEM((8, ns), jnp.int32),
                        pltpu.VMEM((8, ns), jnp.int32)],
    )(xr, yr, zr, cx, cy, cz)
    return out.reshape(b * npoint * ns)


def _sc_gather(table, idx):
    m = idx.shape[0]
    d = table.shape[1]
    sub = 128 if 128 * d * 4 <= 229376 else 64
    groups = m // sub
    per_w = groups // _NW
    idx2 = idx.reshape(groups, sub)
    mesh = plsc.VectorSubcoreMesh(core_axis_name="c", subcore_axis_name="s")

    @functools.partial(
        pl.kernel, mesh=mesh,
        out_type=jax.ShapeDtypeStruct((groups, sub, d), jnp.float32),
        scratch_types=[
            pltpu.VMEM((sub,), jnp.int32),
            pltpu.VMEM((sub, d), jnp.float32),
            pltpu.SemaphoreType.DMA,
        ],
    )
    def k(table_hbm, idx_hbm, out_hbm, idx_v, rows_v, sem):
        wid = lax.axis_index("s") * _NC + lax.axis_index("c")

        def body(g, _):
            grp = wid * per_w + g
            pltpu.sync_copy(idx_hbm.at[grp], idx_v)
            pltpu.async_copy(table_hbm.at[idx_v], rows_v, sem).wait()
            pltpu.sync_copy(rows_v, out_hbm.at[grp])
            return 0

        lax.fori_loop(0, per_w, body, 0)

    return k(table, idx2).reshape(m, d)


def _sa_mlp_kernel(g_ref, c_ref, *refs, cb, ns, nlayer):
    o_ref = refs[3 * nlayer]
    h = (g_ref[...] - c_ref[...]).reshape(cb * ns, g_ref.shape[2])
    for l in range(nlayer):
        w, s, bt = refs[3 * l], refs[3 * l + 1], refs[3 * l + 2]
        h = jnp.dot(h, w[...], preferred_element_type=jnp.float32)
        h = jnp.maximum(h * s[...] + bt[...], 0.0)
    c_out = h.shape[1]
    o_ref[...] = jnp.max(h.reshape(cb, ns, c_out), axis=1)


def _sa_mlp(g, csub, layers, ns, cb):
    rows, d = csub.shape[0], csub.shape[2]
    gr = g.reshape(rows, ns, d)
    wargs = []
    wspecs = []
    for (wt, s, bt) in layers:
        c = wt.shape[1]
        wargs += [wt, s.reshape(1, c), bt.reshape(1, c)]
        wspecs += [
            pl.BlockSpec(wt.shape, lambda i: (0, 0)),
            pl.BlockSpec((1, c), lambda i: (0, 0)),
            pl.BlockSpec((1, c), lambda i: (0, 0)),
        ]
    c_out = layers[-1][0].shape[1]
    return pl.pallas_call(
        functools.partial(_sa_mlp_kernel, cb=cb, ns=ns, nlayer=len(layers)),
        grid=(rows // cb,),
        in_specs=[
            pl.BlockSpec((cb, ns, d), lambda i: (i, 0, 0)),
            pl.BlockSpec((cb, 1, d), lambda i: (i, 0, 0)),
        ] + wspecs,
        out_specs=pl.BlockSpec((cb, c_out), lambda i: (i, 0)),
        out_shape=jax.ShapeDtypeStruct((rows, c_out), jnp.float32),
    )(gr, csub, *wargs)


def _three_nn_kernel(xr, yr, zr, uxr, uyr, uzr, i_ref, w_ref, *, nk):
    b = pl.program_id(0)
    x = xr[0]
    y = yr[0]
    z = zr[0]
    ux = uxr[0, 0]
    uy = uyr[0, 0]
    uz = uzr[0, 0]
    dx = ux - x
    dy = uy - y
    dz = uz - z
    d2 = dx * dx + dy * dy + dz * dz
    kiota = lax.broadcasted_iota(jnp.int32, (8, nk), 1)
    vs, ids = [], []
    for _ in range(3):
        v = jnp.min(d2, axis=1, keepdims=True)
        ii = jnp.min(jnp.where(d2 == v, kiota, nk), axis=1, keepdims=True)
        d2 = jnp.where(kiota == ii, 1e30, d2)
        vs.append(v)
        ids.append(ii)
    r = [1.0 / (jnp.maximum(v, 0.0) + 1e-8) for v in vs]
    rs = r[0] + r[1] + r[2]
    zero = jnp.zeros((8, 1), jnp.float32)
    w_ref[0, 0] = jnp.concatenate([r[0] / rs, r[1] / rs, r[2] / rs, zero],
                                  axis=1)
    izero = jnp.zeros((8, 1), jnp.int32)
    i_ref[0, 0] = jnp.concatenate(ids + [izero], axis=1) + b * nk


def _three_nn(unknown, known):
    b, nu, _ = unknown.shape
    nk = known.shape[1]
    xr = known[..., 0].reshape(b, 1, nk)
    yr = known[..., 1].reshape(b, 1, nk)
    zr = known[..., 2].reshape(b, 1, nk)
    ux = unknown[..., 0].reshape(b, nu // 8, 8, 1)
    uy = unknown[..., 1].reshape(b, nu // 8, 8, 1)
    uz = unknown[..., 2].reshape(b, nu // 8, 8, 1)
    idx4, w4 = pl.pallas_call(
        functools.partial(_three_nn_kernel, nk=nk),
        grid=(b, nu // 8),
        in_specs=[pl.BlockSpec((1, 1, nk), lambda i, j: (i, 0, 0))] * 3
        + [pl.BlockSpec((1, 1, 8, 1), lambda i, j: (i, j, 0, 0))] * 3,
        out_specs=[
            pl.BlockSpec((1, 1, 8, 4), lambda i, j: (i, j, 0, 0)),
            pl.BlockSpec((1, 1, 8, 4), lambda i, j: (i, j, 0, 0)),
        ],
        out_shape=[
            jax.ShapeDtypeStruct((b, nu // 8, 8, 4), jnp.int32),
            jax.ShapeDtypeStruct((b, nu // 8, 8, 4), jnp.float32),
        ],
    )(xr, yr, zr, ux, uy, uz)
    return idx4.reshape(b * nu * 4), w4.reshape(b * nu, 4, 1)


def _fp_mlp_kernel(g_ref, w4_ref, s_ref, *refs, cb, nlayer, final):
    nw = 3 * nlayer + (2 if final else 0)
    o_ref = refs[nw]
    interp = jnp.sum(g_ref[...] * w4_ref[...], axis=1)
    h = jnp.concatenate([interp, s_ref[...]], axis=1)
    for l in range(nlayer):
        w, s, bt = refs[3 * l], refs[3 * l + 1], refs[3 * l + 2]
        h = jnp.dot(h, w[...], preferred_element_type=jnp.float32)
        h = jnp.maximum(h * s[...] + bt[...], 0.0)
    if final:
        wf, bf = refs[3 * nlayer], refs[3 * nlayer + 1]
        h = jnp.dot(h, wf[...], preferred_element_type=jnp.float32) + bf[...]
    o_ref[...] = h


def _fp_mlp(g4, w4, skip, layers, cb, final=None):
    rows = skip.shape[0]
    c = g4.shape[1]
    cs = skip.shape[1]
    gr = g4.reshape(rows, 4, c)
    wargs = []
    wspecs = []
    for (wt, s, bt) in layers:
        co = wt.shape[1]
        wargs += [wt, s.reshape(1, co), bt.reshape(1, co)]
        wspecs += [
            pl.BlockSpec(wt.shape, lambda i: (0, 0)),
            pl.BlockSpec((1, co), lambda i: (0, 0)),
            pl.BlockSpec((1, co), lambda i: (0, 0)),
        ]
    if final is not None:
        wf, bf = final
        co = wf.shape[1]
        wargs += [wf, bf.reshape(1, co)]
        wspecs += [
            pl.BlockSpec(wf.shape, lambda i: (0, 0)),
            pl.BlockSpec((1, co), lambda i: (0, 0)),
        ]
        c_out = co
    else:
        c_out = layers[-1][0].shape[1]
    return pl.pallas_call(
        functools.partial(_fp_mlp_kernel, cb=cb, nlayer=len(layers),
                          final=final is not None),
        grid=(rows // cb,),
        in_specs=[
            pl.BlockSpec((cb, 4, c), lambda i: (i, 0, 0)),
            pl.BlockSpec((cb, 4, 1), lambda i: (i, 0, 0)),
            pl.BlockSpec((cb, cs), lambda i: (i, 0)),
        ] + wspecs,
        out_specs=pl.BlockSpec((cb, c_out), lambda i: (i, 0)),
        out_shape=jax.ShapeDtypeStruct((rows, c_out), jnp.float32),
    )(gr, w4, skip, *wargs)


def _prep_layer(lp, cin_pad=None):
    w = lp["w"]
    if cin_pad is not None and cin_pad > w.shape[1]:
        w = jnp.pad(w, ((0, 0), (0, cin_pad - w.shape[1])))
    scale = lp["gamma"] / np.sqrt(1.0 + _BN_EPS)
    return w.T, scale, lp["beta"]


def _pad_rows(x, d):
    return jnp.pad(x, ((0, 0), (0, d - x.shape[1])))


def _sa_level(xyz, table, d, npoint, radii, nss, scale_params, cbs):
    b, n, _ = xyz.shape
    new_xyz = _fps(xyz, npoint)
    idx1 = _ball(xyz, new_xyz, radii[0], nss[0])
    idx2 = _ball(xyz, new_xyz, radii[1], nss[1])
    rows = _sc_gather(table, jnp.concatenate([idx1, idx2]))
    m1 = idx1.shape[0]
    csub = _pad_rows(new_xyz.reshape(b * npoint, 3), d)[:, None, :]
    outs = []
    for g, ns, lps, cb in ((rows[:m1], nss[0], scale_params[0], cbs),
                           (rows[m1:], nss[1], scale_params[1], cbs)):
        layers = [_prep_layer(lps[0], cin_pad=d)] + [_prep_layer(lp)
                                                     for lp in lps[1:]]
        outs.append(_sa_mlp(g, csub, layers, ns, cb))
    return new_xyz, jnp.concatenate(outs, axis=1)


def _fp_level(unknown, known, feat_known, skip, lps, cb, final=None):
    idx4, w4 = _three_nn(unknown, known)
    g4 = _sc_gather(feat_known, idx4)
    layers = [_prep_layer(lp) for lp in lps]
    return _fp_mlp(g4, w4, skip, layers, cb, final=final)


def kernel(pointcloud, params):
    b, n, _ = pointcloud.shape
    xyz0 = pointcloud[..., 0:3]
    feat0 = pointcloud.reshape(b * n, 9)[:, 3:]

    t1 = _pad_rows(pointcloud.reshape(b * n, 9), 128)
    new1, f1 = _sa_level(xyz0, t1, 128, 4096, (0.4, 0.8), (16, 32),
                         params["sa"][0], 64)
    t2 = _pad_rows(jnp.concatenate([new1.reshape(b * 4096, 3), f1], axis=1),
                   256)
    new2, f2 = _sa_level(new1, t2, 256, 1024, (0.8, 1.2), (16, 32),
                         params["sa"][1], 64)
    t3 = _pad_rows(jnp.concatenate([new2.reshape(b * 1024, 3), f2], axis=1),
                   640)
    new3, f3 = _sa_level(new2, t3, 640, 256, (1.2, 1.6), (16, 32),
                         params["sa"][2], 32)

    fp3 = _fp_level(new2, new3, f3, f2, params["fp"][2], 128)
    fp2 = _fp_level(new1, new2, fp3, f1, params["fp"][1], 256)
    fc = params["fc"]
    final = (fc["w2"].T, fc["b2"])
    lps = list(params["fp"][0]) + [fc["l1"]]
    out = _fp_level(xyz0, new1, fp2, feat0, lps, 512, final=final)
    return out.reshape(b, n, 13)


# speedup vs baseline: 9.4456x; 9.4456x over previous
import functools

import numpy as np
import jax
import jax.numpy as jnp
from jax import lax
from jax.experimental import pallas as pl
from jax.experimental.pallas import tpu as pltpu
from jax.experimental.pallas import tpu_sc as plsc

_BN_EPS = 1e-5
_NC, _NS = 2, 16
_NW = _NC * _NS


def _red2(op, a):
    r = op(a, axis=0, keepdims=True)
    return op(r, axis=1, keepdims=True)


def _fps_kernel(x_ref, y_ref, z_ref, o_ref, d_ref, *, npoint, n):
    nl = n // 8
    x = x_ref[...]
    y = y_ref[...]
    z = z_ref[...]
    i0 = lax.broadcasted_iota(jnp.int32, (16, nl), 0)
    nidx = (i0 % 8) * nl + lax.broadcasted_iota(jnp.int32, (16, nl), 1)
    d_ref[...] = jnp.full((16, nl), 1e10, dtype=jnp.float32)

    def step(t, carry):
        f0, f1 = carry
        farv = jnp.concatenate([jnp.broadcast_to(f0, (8, 1)),
                                jnp.broadcast_to(f1, (8, 1))], axis=0)
        m = nidx == farv
        xm = jnp.where(m, x, 0.0)
        ym = jnp.where(m, y, 0.0)
        zm = jnp.where(m, z, 0.0)
        cx0 = _red2(jnp.sum, xm[0:8])
        cy0 = _red2(jnp.sum, ym[0:8])
        cz0 = _red2(jnp.sum, zm[0:8])
        cx1 = _red2(jnp.sum, xm[8:16])
        cy1 = _red2(jnp.sum, ym[8:16])
        cz1 = _red2(jnp.sum, zm[8:16])
        o_ref[0, pl.ds(t, 1), :] = jnp.concatenate([cx0, cy0, cz0], axis=1)
        o_ref[1, pl.ds(t, 1), :] = jnp.concatenate([cx1, cy1, cz1], axis=1)
        cxv = jnp.concatenate([jnp.broadcast_to(cx0, (8, 1)),
                               jnp.broadcast_to(cx1, (8, 1))], axis=0)
        cyv = jnp.concatenate([jnp.broadcast_to(cy0, (8, 1)),
                               jnp.broadcast_to(cy1, (8, 1))], axis=0)
        czv = jnp.concatenate([jnp.broadcast_to(cz0, (8, 1)),
                               jnp.broadcast_to(cz1, (8, 1))], axis=0)
        dx = x - cxv
        dy = y - cyv
        dz = z - czv
        d = dx * dx + dy * dy + dz * dz
        dn = jnp.minimum(d_ref[...], d)
        d_ref[...] = dn
        mx0 = _red2(jnp.max, dn[0:8])
        mx1 = _red2(jnp.max, dn[8:16])
        nf0 = _red2(jnp.min, jnp.where(dn[0:8] == mx0, nidx[0:8], n))
        nf1 = _red2(jnp.min, jnp.where(dn[8:16] == mx1, nidx[8:16], n))
        return nf0, nf1

    lax.fori_loop(0, npoint, step,
                  (jnp.zeros((1, 1), jnp.int32), jnp.zeros((1, 1), jnp.int32)))


def _fps(xyz, npoint):
    b, n, _ = xyz.shape
    nl = n // 8
    xs = xyz[..., 0].reshape(b * 8, nl)
    ys = xyz[..., 1].reshape(b * 8, nl)
    zs = xyz[..., 2].reshape(b * 8, nl)
    return pl.pallas_call(
        functools.partial(_fps_kernel, npoint=npoint, n=n),
        out_shape=jax.ShapeDtypeStruct((b, npoint, 3), jnp.float32),
        scratch_shapes=[pltpu.VMEM((16, nl), jnp.float32)],
    )(xs, ys, zs)


def _shr_last(a, sh):
    pad = [(0, 0)] * (a.ndim - 1) + [(sh, 0)]
    return jnp.pad(a, pad)[..., : a.shape[-1]]


def _shr_mid(a, sh):
    return jnp.pad(a, ((0, 0), (sh, 0), (0, 0)))[:, : a.shape[1], :]


def _mask_rank(mask, n):
    m = mask.astype(jnp.int32)
    c = m
    sh = 1
    while sh < n:
        c = c + _shr_last(c, sh)
        sh *= 2
    return c - m


def _ball_kernel(xr, yr, zr, cxr, cyr, czr, o_ref, *, r2, ns, n):
    b = pl.program_id(0)
    x = xr[0]
    y = yr[0]
    z = zr[0]
    cx = cxr[0, 0]
    cy = cyr[0, 0]
    cz = czr[0, 0]
    dx = cx - x
    dy = cy - y
    dz = cz - z
    d2 = dx * dx + dy * dy + dz * dz
    mask = d2 <= r2
    rank = jnp.where(mask, _mask_rank(mask, n), -1)
    kiota = lax.broadcasted_iota(jnp.int32, (8, n), 1)
    cols = [jnp.min(jnp.where(rank == s, kiota, n), axis=1, keepdims=True)
            for s in range(ns)]
    idx = jnp.concatenate(cols, axis=1)
    first = idx[:, 0:1]
    idx = jnp.where(idx == n, first, idx)
    idx = jnp.where(idx == n, 0, idx)
    o_ref[0, 0] = idx + b * n


def _ball(xyz, new_xyz, radius, ns):
    b, n, _ = xyz.shape
    npoint = new_xyz.shape[1]
    xr = xyz[..., 0].reshape(b, 1, n)
    yr = xyz[..., 1].reshape(b, 1, n)
    zr = xyz[..., 2].reshape(b, 1, n)
    cx = new_xyz[..., 0].reshape(b, npoint // 8, 8, 1)
    cy = new_xyz[..., 1].reshape(b, npoint // 8, 8, 1)
    cz = new_xyz[..., 2].reshape(b, npoint // 8, 8, 1)
    out = pl.pallas_call(
        functools.partial(_ball_kernel, r2=radius * radius, ns=ns, n=n),
        grid=(b, npoint // 8),
        in_specs=[pl.BlockSpec((1, 1, n), lambda i, j: (i, 0, 0))] * 3
        + [pl.BlockSpec((1, 1, 8, 1), lambda i, j: (i, j, 0, 0))] * 3,
        out_specs=pl.BlockSpec((1, 1, 8, ns), lambda i, j: (i, j, 0, 0)),
        out_shape=jax.ShapeDtypeStruct((b, npoint // 8, 8, ns), jnp.int32),
    )(xr, yr, zr, cx, cy, cz)
    return out.reshape(b * npoint * ns)


def _sc_gather(table, idx):
    m = idx.shape[0]
    d = table.shape[1]
    sub = 128 if 128 * d * 4 <= 229376 else 64
    groups = m // sub
    per_w = groups // _NW
    idx2 = idx.reshape(groups, sub)
    mesh = plsc.VectorSubcoreMesh(core_axis_name="c", subcore_axis_name="s")

    @functools.partial(
        pl.kernel, mesh=mesh,
        out_type=jax.ShapeDtypeStruct((groups, sub, d), jnp.float32),
        scratch_types=[
            pltpu.VMEM((sub,), jnp.int32),
            pltpu.VMEM((sub, d), jnp.float32),
            pltpu.SemaphoreType.DMA,
        ],
    )
    def k(table_hbm, idx_hbm, out_hbm, idx_v, rows_v, sem):
        wid = lax.axis_index("s") * _NC + lax.axis_index("c")

        def body(g, _):
            grp = wid * per_w + g
            pltpu.sync_copy(idx_hbm.at[grp], idx_v)
            pltpu.async_copy(table_hbm.at[idx_v], rows_v, sem).wait()
            pltpu.sync_copy(rows_v, out_hbm.at[grp])
            return 0

        lax.fori_loop(0, per_w, body, 0)

    return k(table, idx2).reshape(m, d)


def _sa_mlp_kernel(g_ref, c_ref, *refs, cb, ns, nlayer):
    o_ref = refs[3 * nlayer]
    h = (g_ref[...] - c_ref[...]).reshape(cb * ns, g_ref.shape[2])
    for l in range(nlayer):
        w, s, bt = refs[3 * l], refs[3 * l + 1], refs[3 * l + 2]
        h = jnp.dot(h, w[...], preferred_element_type=jnp.float32)
        h = jnp.maximum(h * s[...] + bt[...], 0.0)
    c_out = h.shape[1]
    o_ref[...] = jnp.max(h.reshape(cb, ns, c_out), axis=1)


def _sa_mlp(g, csub, layers, ns, cb):
    rows, d = csub.shape[0], csub.shape[2]
    gr = g.reshape(rows, ns, d)
    wargs = []
    wspecs = []
    for (wt, s, bt) in layers:
        c = wt.shape[1]
        wargs += [wt, s.reshape(1, c), bt.reshape(1, c)]
        wspecs += [
            pl.BlockSpec(wt.shape, lambda i: (0, 0)),
            pl.BlockSpec((1, c), lambda i: (0, 0)),
            pl.BlockSpec((1, c), lambda i: (0, 0)),
        ]
    c_out = layers[-1][0].shape[1]
    return pl.pallas_call(
        functools.partial(_sa_mlp_kernel, cb=cb, ns=ns, nlayer=len(layers)),
        grid=(rows // cb,),
        in_specs=[
            pl.BlockSpec((cb, ns, d), lambda i: (i, 0, 0)),
            pl.BlockSpec((cb, 1, d), lambda i: (i, 0, 0)),
        ] + wspecs,
        out_specs=pl.BlockSpec((cb, c_out), lambda i: (i, 0)),
        out_shape=jax.ShapeDtypeStruct((rows, c_out), jnp.float32),
    )(gr, csub, *wargs)


def _three_nn_kernel(xr, yr, zr, uxr, uyr, uzr, i_ref, w_ref, *, nk):
    b = pl.program_id(0)
    x = xr[0]
    y = yr[0]
    z = zr[0]
    ux = uxr[0, 0]
    uy = uyr[0, 0]
    uz = uzr[0, 0]
    dx = ux - x
    dy = uy - y
    dz = uz - z
    d2 = dx * dx + dy * dy + dz * dz
    kiota = lax.broadcasted_iota(jnp.int32, (8, nk), 1)
    vs, ids = [], []
    for _ in range(3):
        v = jnp.min(d2, axis=1, keepdims=True)
        ii = jnp.min(jnp.where(d2 == v, kiota, nk), axis=1, keepdims=True)
        d2 = jnp.where(kiota == ii, 1e30, d2)
        vs.append(v)
        ids.append(ii)
    r = [1.0 / (jnp.maximum(v, 0.0) + 1e-8) for v in vs]
    rs = r[0] + r[1] + r[2]
    zero = jnp.zeros((8, 1), jnp.float32)
    w_ref[0, 0] = jnp.concatenate([r[0] / rs, r[1] / rs, r[2] / rs, zero],
                                  axis=1)
    izero = jnp.zeros((8, 1), jnp.int32)
    i_ref[0, 0] = jnp.concatenate(ids + [izero], axis=1) + b * nk


def _three_nn(unknown, known):
    b, nu, _ = unknown.shape
    nk = known.shape[1]
    xr = known[..., 0].reshape(b, 1, nk)
    yr = known[..., 1].reshape(b, 1, nk)
    zr = known[..., 2].reshape(b, 1, nk)
    ux = unknown[..., 0].reshape(b, nu // 8, 8, 1)
    uy = unknown[..., 1].reshape(b, nu // 8, 8, 1)
    uz = unknown[..., 2].reshape(b, nu // 8, 8, 1)
    idx4, w4 = pl.pallas_call(
        functools.partial(_three_nn_kernel, nk=nk),
        grid=(b, nu // 8),
        in_specs=[pl.BlockSpec((1, 1, nk), lambda i, j: (i, 0, 0))] * 3
        + [pl.BlockSpec((1, 1, 8, 1), lambda i, j: (i, j, 0, 0))] * 3,
        out_specs=[
            pl.BlockSpec((1, 1, 8, 4), lambda i, j: (i, j, 0, 0)),
            pl.BlockSpec((1, 1, 8, 4), lambda i, j: (i, j, 0, 0)),
        ],
        out_shape=[
            jax.ShapeDtypeStruct((b, nu // 8, 8, 4), jnp.int32),
            jax.ShapeDtypeStruct((b, nu // 8, 8, 4), jnp.float32),
        ],
    )(xr, yr, zr, ux, uy, uz)
    return idx4.reshape(b * nu * 4), w4.reshape(b * nu, 4, 1)


def _fp_mlp_kernel(g_ref, w4_ref, s_ref, *refs, cb, nlayer, final):
    nw = 3 * nlayer + (2 if final else 0)
    o_ref = refs[nw]
    interp = jnp.sum(g_ref[...] * w4_ref[...], axis=1)
    h = jnp.concatenate([interp, s_ref[...]], axis=1)
    for l in range(nlayer):
        w, s, bt = refs[3 * l], refs[3 * l + 1], refs[3 * l + 2]
        h = jnp.dot(h, w[...], preferred_element_type=jnp.float32)
        h = jnp.maximum(h * s[...] + bt[...], 0.0)
    if final:
        wf, bf = refs[3 * nlayer], refs[3 * nlayer + 1]
        h = jnp.dot(h, wf[...], preferred_element_type=jnp.float32) + bf[...]
    o_ref[...] = h


def _fp_mlp(g4, w4, skip, layers, cb, final=None):
    rows = skip.shape[0]
    c = g4.shape[1]
    cs = skip.shape[1]
    gr = g4.reshape(rows, 4, c)
    wargs = []
    wspecs = []
    for (wt, s, bt) in layers:
        co = wt.shape[1]
        wargs += [wt, s.reshape(1, co), bt.reshape(1, co)]
        wspecs += [
            pl.BlockSpec(wt.shape, lambda i: (0, 0)),
            pl.BlockSpec((1, co), lambda i: (0, 0)),
            pl.BlockSpec((1, co), lambda i: (0, 0)),
        ]
    if final is not None:
        wf, bf = final
        co = wf.shape[1]
        wargs += [wf, bf.reshape(1, co)]
        wspecs += [
            pl.BlockSpec(wf.shape, lambda i: (0, 0)),
            pl.BlockSpec((1, co), lambda i: (0, 0)),
        ]
        c_out = co
    else:
        c_out = layers[-1][0].shape[1]
    return pl.pallas_call(
        functools.partial(_fp_mlp_kernel, cb=cb, nlayer=len(layers),
                          final=final is not None),
        grid=(rows // cb,),
        in_specs=[
            pl.BlockSpec((cb, 4, c), lambda i: (i, 0, 0)),
            pl.BlockSpec((cb, 4, 1), lambda i: (i, 0, 0)),
            pl.BlockSpec((cb, cs), lambda i: (i, 0)),
        ] + wspecs,
        out_specs=pl.BlockSpec((cb, c_out), lambda i: (i, 0)),
        out_shape=jax.ShapeDtypeStruct((rows, c_out), jnp.float32),
    )(gr, w4, skip, *wargs)


def _prep_layer(lp, cin_pad=None):
    w = lp["w"]
    if cin_pad is not None and cin_pad > w.shape[1]:
        w = jnp.pad(w, ((0, 0), (0, cin_pad - w.shape[1])))
    scale = lp["gamma"] / np.sqrt(1.0 + _BN_EPS)
    return w.T, scale, lp["beta"]


def _pad_rows(x, d):
    return jnp.pad(x, ((0, 0), (0, d - x.shape[1])))


def _sa_level(xyz, table, d, npoint, radii, nss, scale_params, cbs):
    b, n, _ = xyz.shape
    new_xyz = _fps(xyz, npoint)
    idx1 = _ball(xyz, new_xyz, radii[0], nss[0])
    idx2 = _ball(xyz, new_xyz, radii[1], nss[1])
    rows = _sc_gather(table, jnp.concatenate([idx1, idx2]))
    m1 = idx1.shape[0]
    csub = _pad_rows(new_xyz.reshape(b * npoint, 3), d)[:, None, :]
    outs = []
    for g, ns, lps, cb in ((rows[:m1], nss[0], scale_params[0], cbs),
                           (rows[m1:], nss[1], scale_params[1], cbs)):
        layers = [_prep_layer(lps[0], cin_pad=d)] + [_prep_layer(lp)
                                                     for lp in lps[1:]]
        outs.append(_sa_mlp(g, csub, layers, ns, cb))
    return new_xyz, jnp.concatenate(outs, axis=1)


def _fp_level(unknown, known, feat_known, skip, lps, cb, final=None):
    idx4, w4 = _three_nn(unknown, known)
    g4 = _sc_gather(feat_known, idx4)
    layers = [_prep_layer(lp) for lp in lps]
    return _fp_mlp(g4, w4, skip, layers, cb, final=final)


def kernel(pointcloud, params):
    b, n, _ = pointcloud.shape
    xyz0 = pointcloud[..., 0:3]
    feat0 = pointcloud.reshape(b * n, 9)[:, 3:]

    t1 = _pad_rows(pointcloud.reshape(b * n, 9), 128)
    new1, f1 = _sa_level(xyz0, t1, 128, 4096, (0.4, 0.8), (16, 32),
                         params["sa"][0], 64)
    t2 = _pad_rows(jnp.concatenate([new1.reshape(b * 4096, 3), f1], axis=1),
                   256)
    new2, f2 = _sa_level(new1, t2, 256, 1024, (0.8, 1.2), (16, 32),
                         params["sa"][1], 64)
    t3 = _pad_rows(jnp.concatenate([new2.reshape(b * 1024, 3), f2], axis=1),
                   640)
    new3, f3 = _sa_level(new2, t3, 640, 256, (1.2, 1.6), (16, 32),
                         params["sa"][2], 32)

    fp3 = _fp_level(new2, new3, f3, f2, params["fp"][2], 128)
    fp2 = _fp_level(new1, new2, fp3, f1, params["fp"][1], 256)
    fc = params["fc"]
    final = (fc["w2"].T, fc["b2"])
    lps = list(params["fp"][0]) + [fc["l1"]]
    out = _fp_level(xyz0, new1, fp2, feat0, lps, 512, final=final)
    return out.reshape(b, n, 13)


# speedup vs baseline: 13.0086x; 1.3772x over previous
import functools

import numpy as np
import jax
import jax.numpy as jnp
from jax import lax
from jax.experimental import pallas as pl
from jax.experimental.pallas import tpu as pltpu
from jax.experimental.pallas import tpu_sc as plsc

_BN_EPS = 1e-5
_NC, _NS = 2, 16
_NW = _NC * _NS


def _red2(op, a):
    r = op(a, axis=0, keepdims=True)
    return op(r, axis=1, keepdims=True)


def _fps_kernel(x_ref, y_ref, z_ref, o_ref, d_ref, *, npoint, n):
    nl = n // 8
    x = x_ref[...]
    y = y_ref[...]
    z = z_ref[...]
    i0 = lax.broadcasted_iota(jnp.int32, (16, nl), 0)
    nidx = (i0 % 8) * nl + lax.broadcasted_iota(jnp.int32, (16, nl), 1)
    d_ref[...] = jnp.full((16, nl), 1e10, dtype=jnp.float32)

    def step(t, carry):
        f0, f1 = carry
        farv = jnp.concatenate([jnp.broadcast_to(f0, (8, 1)),
                                jnp.broadcast_to(f1, (8, 1))], axis=0)
        m = nidx == farv
        xm = jnp.where(m, x, 0.0)
        ym = jnp.where(m, y, 0.0)
        zm = jnp.where(m, z, 0.0)
        cx0 = _red2(jnp.sum, xm[0:8])
        cy0 = _red2(jnp.sum, ym[0:8])
        cz0 = _red2(jnp.sum, zm[0:8])
        cx1 = _red2(jnp.sum, xm[8:16])
        cy1 = _red2(jnp.sum, ym[8:16])
        cz1 = _red2(jnp.sum, zm[8:16])
        o_ref[0, pl.ds(t, 1), :] = jnp.concatenate([cx0, cy0, cz0], axis=1)
        o_ref[1, pl.ds(t, 1), :] = jnp.concatenate([cx1, cy1, cz1], axis=1)
        cxv = jnp.concatenate([jnp.broadcast_to(cx0, (8, 1)),
                               jnp.broadcast_to(cx1, (8, 1))], axis=0)
        cyv = jnp.concatenate([jnp.broadcast_to(cy0, (8, 1)),
                               jnp.broadcast_to(cy1, (8, 1))], axis=0)
        czv = jnp.concatenate([jnp.broadcast_to(cz0, (8, 1)),
                               jnp.broadcast_to(cz1, (8, 1))], axis=0)
        dx = x - cxv
        dy = y - cyv
        dz = z - czv
        d = dx * dx + dy * dy + dz * dz
        dn = jnp.minimum(d_ref[...], d)
        d_ref[...] = dn
        mx0 = _red2(jnp.max, dn[0:8])
        mx1 = _red2(jnp.max, dn[8:16])
        nf0 = _red2(jnp.min, jnp.where(dn[0:8] == mx0, nidx[0:8], n))
        nf1 = _red2(jnp.min, jnp.where(dn[8:16] == mx1, nidx[8:16], n))
        return nf0, nf1

    lax.fori_loop(0, npoint, step,
                  (jnp.zeros((1, 1), jnp.int32), jnp.zeros((1, 1), jnp.int32)))


def _fps(xyz, npoint):
    b, n, _ = xyz.shape
    nl = n // 8
    xs = xyz[..., 0].reshape(b * 8, nl)
    ys = xyz[..., 1].reshape(b * 8, nl)
    zs = xyz[..., 2].reshape(b * 8, nl)
    return pl.pallas_call(
        functools.partial(_fps_kernel, npoint=npoint, n=n),
        out_shape=jax.ShapeDtypeStruct((b, npoint, 3), jnp.float32),
        scratch_shapes=[pltpu.VMEM((16, nl), jnp.float32)],
    )(xs, ys, zs)


def _shr_last(a, sh):
    pad = [(0, 0)] * (a.ndim - 1) + [(sh, 0)]
    return jnp.pad(a, pad)[..., : a.shape[-1]]


def _shr_mid(a, sh):
    return jnp.pad(a, ((0, 0), (sh, 0), (0, 0)))[:, : a.shape[1], :]


def _mask_rank(mask, n):
    m = mask.astype(jnp.int32)
    c = m
    sh = 1
    while sh < n:
        c = c + _shr_last(c, sh)
        sh *= 2
    return c - m


def _ball_kernel(xr, yr, zr, cxr, cyr, czr, o_ref, *, r2, ns, n, rb):
    b = pl.program_id(0)
    x = xr[0]
    y = yr[0]
    z = zr[0]
    cx = cxr[0, 0]
    cy = cyr[0, 0]
    cz = czr[0, 0]
    dx = cx - x
    dy = cy - y
    dz = cz - z
    d2 = dx * dx + dy * dy + dz * dz
    mask = d2 <= r2
    rank = jnp.where(mask, _mask_rank(mask, n), -1)
    kiota = lax.broadcasted_iota(jnp.int32, (rb, n), 1)
    cols = [jnp.min(jnp.where(rank == s, kiota, n), axis=1, keepdims=True)
            for s in range(ns)]
    idx = jnp.concatenate(cols, axis=1)
    first = idx[:, 0:1]
    idx = jnp.where(idx == n, first, idx)
    idx = jnp.where(idx == n, 0, idx)
    o_ref[0, 0] = idx + b * n


def _ball(xyz, new_xyz, radius, ns):
    b, n, _ = xyz.shape
    npoint = new_xyz.shape[1]
    rb = min(32, npoint)
    xr = xyz[..., 0].reshape(b, 1, n)
    yr = xyz[..., 1].reshape(b, 1, n)
    zr = xyz[..., 2].reshape(b, 1, n)
    cx = new_xyz[..., 0].reshape(b, npoint // rb, rb, 1)
    cy = new_xyz[..., 1].reshape(b, npoint // rb, rb, 1)
    cz = new_xyz[..., 2].reshape(b, npoint // rb, rb, 1)
    out = pl.pallas_call(
        functools.partial(_ball_kernel, r2=radius * radius, ns=ns, n=n,
                          rb=rb),
        grid=(b, npoint // rb),
        in_specs=[pl.BlockSpec((1, 1, n), lambda i, j: (i, 0, 0))] * 3
        + [pl.BlockSpec((1, 1, rb, 1), lambda i, j: (i, j, 0, 0))] * 3,
        out_specs=pl.BlockSpec((1, 1, rb, ns), lambda i, j: (i, j, 0, 0)),
        out_shape=jax.ShapeDtypeStruct((b, npoint // rb, rb, ns), jnp.int32),
    )(xr, yr, zr, cx, cy, cz)
    return out.reshape(b * npoint * ns)


def _sc_gather(table, idx):
    m = idx.shape[0]
    d = table.shape[1]
    sub = 128 if 128 * d * 4 <= 229376 else 64
    groups = m // sub
    per_w = groups // _NW
    idx2 = idx.reshape(groups, sub)
    mesh = plsc.VectorSubcoreMesh(core_axis_name="c", subcore_axis_name="s")

    @functools.partial(
        pl.kernel, mesh=mesh,
        out_type=jax.ShapeDtypeStruct((groups, sub, d), jnp.float32),
        scratch_types=[
            pltpu.VMEM((sub,), jnp.int32),
            pltpu.VMEM((sub, d), jnp.float32),
            pltpu.SemaphoreType.DMA,
        ],
    )
    def k(table_hbm, idx_hbm, out_hbm, idx_v, rows_v, sem):
        wid = lax.axis_index("s") * _NC + lax.axis_index("c")

        def body(g, _):
            grp = wid * per_w + g
            pltpu.sync_copy(idx_hbm.at[grp], idx_v)
            pltpu.async_copy(table_hbm.at[idx_v], rows_v, sem).wait()
            pltpu.sync_copy(rows_v, out_hbm.at[grp])
            return 0

        lax.fori_loop(0, per_w, body, 0)

    return k(table, idx2).reshape(m, d)


def _sa_mlp_kernel(g_ref, c_ref, *refs, cb, ns, nlayer):
    o_ref = refs[3 * nlayer]
    h = (g_ref[...] - c_ref[...]).reshape(cb * ns, g_ref.shape[2])
    for l in range(nlayer):
        w, s, bt = refs[3 * l], refs[3 * l + 1], refs[3 * l + 2]
        h = jnp.dot(h, w[...], preferred_element_type=jnp.float32)
        h = jnp.maximum(h * s[...] + bt[...], 0.0)
    c_out = h.shape[1]
    o_ref[...] = jnp.max(h.reshape(cb, ns, c_out), axis=1)


def _sa_mlp(g, csub, layers, ns, cb):
    rows, d = csub.shape[0], csub.shape[2]
    gr = g.reshape(rows, ns, d)
    wargs = []
    wspecs = []
    for (wt, s, bt) in layers:
        c = wt.shape[1]
        wargs += [wt, s.reshape(1, c), bt.reshape(1, c)]
        wspecs += [
            pl.BlockSpec(wt.shape, lambda i: (0, 0)),
            pl.BlockSpec((1, c), lambda i: (0, 0)),
            pl.BlockSpec((1, c), lambda i: (0, 0)),
        ]
    c_out = layers[-1][0].shape[1]
    return pl.pallas_call(
        functools.partial(_sa_mlp_kernel, cb=cb, ns=ns, nlayer=len(layers)),
        grid=(rows // cb,),
        in_specs=[
            pl.BlockSpec((cb, ns, d), lambda i: (i, 0, 0)),
            pl.BlockSpec((cb, 1, d), lambda i: (i, 0, 0)),
        ] + wspecs,
        out_specs=pl.BlockSpec((cb, c_out), lambda i: (i, 0)),
        out_shape=jax.ShapeDtypeStruct((rows, c_out), jnp.float32),
    )(gr, csub, *wargs)


def _three_nn_kernel(xr, yr, zr, uxr, uyr, uzr, i_ref, w_ref, *, nk, rb):
    b = pl.program_id(0)
    x = xr[0]
    y = yr[0]
    z = zr[0]
    ux = uxr[0, 0]
    uy = uyr[0, 0]
    uz = uzr[0, 0]
    dx = ux - x
    dy = uy - y
    dz = uz - z
    d2 = dx * dx + dy * dy + dz * dz
    kiota = lax.broadcasted_iota(jnp.int32, (rb, nk), 1)
    vs, ids = [], []
    for _ in range(3):
        v = jnp.min(d2, axis=1, keepdims=True)
        ii = jnp.min(jnp.where(d2 == v, kiota, nk), axis=1, keepdims=True)
        d2 = jnp.where(kiota == ii, 1e30, d2)
        vs.append(v)
        ids.append(ii)
    r = [1.0 / (jnp.maximum(v, 0.0) + 1e-8) for v in vs]
    rs = r[0] + r[1] + r[2]
    zero = jnp.zeros((rb, 1), jnp.float32)
    w_ref[0, 0] = jnp.concatenate([r[0] / rs, r[1] / rs, r[2] / rs, zero],
                                  axis=1)
    izero = jnp.zeros((rb, 1), jnp.int32)
    i_ref[0, 0] = jnp.concatenate(ids + [izero], axis=1) + b * nk


def _three_nn(unknown, known):
    b, nu, _ = unknown.shape
    nk = known.shape[1]
    rb = min(64, nu)
    xr = known[..., 0].reshape(b, 1, nk)
    yr = known[..., 1].reshape(b, 1, nk)
    zr = known[..., 2].reshape(b, 1, nk)
    ux = unknown[..., 0].reshape(b, nu // rb, rb, 1)
    uy = unknown[..., 1].reshape(b, nu // rb, rb, 1)
    uz = unknown[..., 2].reshape(b, nu // rb, rb, 1)
    idx4, w4 = pl.pallas_call(
        functools.partial(_three_nn_kernel, nk=nk, rb=rb),
        grid=(b, nu // rb),
        in_specs=[pl.BlockSpec((1, 1, nk), lambda i, j: (i, 0, 0))] * 3
        + [pl.BlockSpec((1, 1, rb, 1), lambda i, j: (i, j, 0, 0))] * 3,
        out_specs=[
            pl.BlockSpec((1, 1, rb, 4), lambda i, j: (i, j, 0, 0)),
            pl.BlockSpec((1, 1, rb, 4), lambda i, j: (i, j, 0, 0)),
        ],
        out_shape=[
            jax.ShapeDtypeStruct((b, nu // rb, rb, 4), jnp.int32),
            jax.ShapeDtypeStruct((b, nu // rb, rb, 4), jnp.float32),
        ],
    )(xr, yr, zr, ux, uy, uz)
    return idx4.reshape(b * nu * 4), w4.reshape(b * nu, 4, 1)


def _fp_mlp_kernel(g_ref, w4_ref, s_ref, *refs, cb, nlayer, final):
    nw = 3 * nlayer + (2 if final else 0)
    o_ref = refs[nw]
    interp = jnp.sum(g_ref[...] * w4_ref[...], axis=1)
    h = jnp.concatenate([interp, s_ref[...]], axis=1)
    for l in range(nlayer):
        w, s, bt = refs[3 * l], refs[3 * l + 1], refs[3 * l + 2]
        h = jnp.dot(h, w[...], preferred_element_type=jnp.float32)
        h = jnp.maximum(h * s[...] + bt[...], 0.0)
    if final:
        wf, bf = refs[3 * nlayer], refs[3 * nlayer + 1]
        h = jnp.dot(h, wf[...], preferred_element_type=jnp.float32) + bf[...]
    o_ref[...] = h


def _fp_mlp(g4, w4, skip, layers, cb, final=None):
    rows = skip.shape[0]
    c = g4.shape[1]
    cs = skip.shape[1]
    gr = g4.reshape(rows, 4, c)
    wargs = []
    wspecs = []
    for (wt, s, bt) in layers:
        co = wt.shape[1]
        wargs += [wt, s.reshape(1, co), bt.reshape(1, co)]
        wspecs += [
            pl.BlockSpec(wt.shape, lambda i: (0, 0)),
            pl.BlockSpec((1, co), lambda i: (0, 0)),
            pl.BlockSpec((1, co), lambda i: (0, 0)),
        ]
    if final is not None:
        wf, bf = final
        co = wf.shape[1]
        wargs += [wf, bf.reshape(1, co)]
        wspecs += [
            pl.BlockSpec(wf.shape, lambda i: (0, 0)),
            pl.BlockSpec((1, co), lambda i: (0, 0)),
        ]
        c_out = co
    else:
        c_out = layers[-1][0].shape[1]
    return pl.pallas_call(
        functools.partial(_fp_mlp_kernel, cb=cb, nlayer=len(layers),
                          final=final is not None),
        grid=(rows // cb,),
        in_specs=[
            pl.BlockSpec((cb, 4, c), lambda i: (i, 0, 0)),
            pl.BlockSpec((cb, 4, 1), lambda i: (i, 0, 0)),
            pl.BlockSpec((cb, cs), lambda i: (i, 0)),
        ] + wspecs,
        out_specs=pl.BlockSpec((cb, c_out), lambda i: (i, 0)),
        out_shape=jax.ShapeDtypeStruct((rows, c_out), jnp.float32),
    )(gr, w4, skip, *wargs)


def _prep_layer(lp, cin_pad=None):
    w = lp["w"]
    if cin_pad is not None and cin_pad > w.shape[1]:
        w = jnp.pad(w, ((0, 0), (0, cin_pad - w.shape[1])))
    scale = lp["gamma"] / np.sqrt(1.0 + _BN_EPS)
    return w.T, scale, lp["beta"]


def _pad_rows(x, d):
    return jnp.pad(x, ((0, 0), (0, d - x.shape[1])))


def _sa_level(xyz, table, d, npoint, radii, nss, scale_params, cbs):
    b, n, _ = xyz.shape
    new_xyz = _fps(xyz, npoint)
    idx1 = _ball(xyz, new_xyz, radii[0], nss[0])
    idx2 = _ball(xyz, new_xyz, radii[1], nss[1])
    rows = _sc_gather(table, jnp.concatenate([idx1, idx2]))
    m1 = idx1.shape[0]
    csub = _pad_rows(new_xyz.reshape(b * npoint, 3), d)[:, None, :]
    outs = []
    for g, ns, lps, cb in ((rows[:m1], nss[0], scale_params[0], cbs),
                           (rows[m1:], nss[1], scale_params[1], cbs)):
        layers = [_prep_layer(lps[0], cin_pad=d)] + [_prep_layer(lp)
                                                     for lp in lps[1:]]
        outs.append(_sa_mlp(g, csub, layers, ns, cb))
    return new_xyz, jnp.concatenate(outs, axis=1)


def _fp_level(unknown, known, feat_known, skip, lps, cb, final=None):
    idx4, w4 = _three_nn(unknown, known)
    g4 = _sc_gather(feat_known, idx4)
    layers = [_prep_layer(lp) for lp in lps]
    return _fp_mlp(g4, w4, skip, layers, cb, final=final)


def kernel(pointcloud, params):
    b, n, _ = pointcloud.shape
    xyz0 = pointcloud[..., 0:3]
    feat0 = pointcloud.reshape(b * n, 9)[:, 3:]

    t1 = _pad_rows(pointcloud.reshape(b * n, 9), 128)
    new1, f1 = _sa_level(xyz0, t1, 128, 4096, (0.4, 0.8), (16, 32),
                         params["sa"][0], 64)
    t2 = _pad_rows(jnp.concatenate([new1.reshape(b * 4096, 3), f1], axis=1),
                   256)
    new2, f2 = _sa_level(new1, t2, 256, 1024, (0.8, 1.2), (16, 32),
                         params["sa"][1], 64)
    t3 = _pad_rows(jnp.concatenate([new2.reshape(b * 1024, 3), f2], axis=1),
                   640)
    new3, f3 = _sa_level(new2, t3, 640, 256, (1.2, 1.6), (16, 32),
                         params["sa"][2], 32)

    fp3 = _fp_level(new2, new3, f3, f2, params["fp"][2], 128)
    fp2 = _fp_level(new1, new2, fp3, f1, params["fp"][1], 256)
    fc = params["fc"]
    final = (fc["w2"].T, fc["b2"])
    lps = list(params["fp"][0]) + [fc["l1"]]
    out = _fp_level(xyz0, new1, fp2, feat0, lps, 512, final=final)
    return out.reshape(b, n, 13)


# speedup vs baseline: 13.0198x; 1.0009x over previous
import functools

import numpy as np
import jax
import jax.numpy as jnp
from jax import lax
from jax.experimental import pallas as pl
from jax.experimental.pallas import tpu as pltpu
from jax.experimental.pallas import tpu_sc as plsc

_BN_EPS = 1e-5
_NC, _NS = 2, 16
_NW = _NC * _NS


def _red2(op, a):
    r = op(a, axis=0, keepdims=True)
    return op(r, axis=1, keepdims=True)


def _fps_kernel(x_ref, y_ref, z_ref, o_ref, d_ref, *, npoint, n):
    nl = n // 8
    x = x_ref[...]
    y = y_ref[...]
    z = z_ref[...]
    i0 = lax.broadcasted_iota(jnp.int32, (16, nl), 0)
    nidx = (i0 % 8) * nl + lax.broadcasted_iota(jnp.int32, (16, nl), 1)
    d_ref[...] = jnp.full((16, nl), 1e10, dtype=jnp.float32)

    def step(t, carry):
        f0, f1 = carry
        farv = jnp.concatenate([jnp.broadcast_to(f0, (8, 1)),
                                jnp.broadcast_to(f1, (8, 1))], axis=0)
        m = nidx == farv
        xm = jnp.where(m, x, 0.0)
        ym = jnp.where(m, y, 0.0)
        zm = jnp.where(m, z, 0.0)
        cx0 = _red2(jnp.sum, xm[0:8])
        cy0 = _red2(jnp.sum, ym[0:8])
        cz0 = _red2(jnp.sum, zm[0:8])
        cx1 = _red2(jnp.sum, xm[8:16])
        cy1 = _red2(jnp.sum, ym[8:16])
        cz1 = _red2(jnp.sum, zm[8:16])
        o_ref[0, pl.ds(t, 1), :] = jnp.concatenate([cx0, cy0, cz0], axis=1)
        o_ref[1, pl.ds(t, 1), :] = jnp.concatenate([cx1, cy1, cz1], axis=1)
        cxv = jnp.concatenate([jnp.broadcast_to(cx0, (8, 1)),
                               jnp.broadcast_to(cx1, (8, 1))], axis=0)
        cyv = jnp.concatenate([jnp.broadcast_to(cy0, (8, 1)),
                               jnp.broadcast_to(cy1, (8, 1))], axis=0)
        czv = jnp.concatenate([jnp.broadcast_to(cz0, (8, 1)),
                               jnp.broadcast_to(cz1, (8, 1))], axis=0)
        dx = x - cxv
        dy = y - cyv
        dz = z - czv
        d = dx * dx + dy * dy + dz * dz
        dn = jnp.minimum(d_ref[...], d)
        d_ref[...] = dn
        mx0 = _red2(jnp.max, dn[0:8])
        mx1 = _red2(jnp.max, dn[8:16])
        nf0 = _red2(jnp.min, jnp.where(dn[0:8] == mx0, nidx[0:8], n))
        nf1 = _red2(jnp.min, jnp.where(dn[8:16] == mx1, nidx[8:16], n))
        return nf0, nf1

    lax.fori_loop(0, npoint, step,
                  (jnp.zeros((1, 1), jnp.int32), jnp.zeros((1, 1), jnp.int32)))


def _fps(xyz, npoint):
    b, n, _ = xyz.shape
    nl = n // 8
    xs = xyz[..., 0].reshape(b * 8, nl)
    ys = xyz[..., 1].reshape(b * 8, nl)
    zs = xyz[..., 2].reshape(b * 8, nl)
    return pl.pallas_call(
        functools.partial(_fps_kernel, npoint=npoint, n=n),
        out_shape=jax.ShapeDtypeStruct((b, npoint, 3), jnp.float32),
        scratch_shapes=[pltpu.VMEM((16, nl), jnp.float32)],
    )(xs, ys, zs)


def _shr_last(a, sh):
    pad = [(0, 0)] * (a.ndim - 1) + [(sh, 0)]
    return jnp.pad(a, pad)[..., : a.shape[-1]]


def _shr_mid(a, sh):
    return jnp.pad(a, ((0, 0), (sh, 0), (0, 0)))[:, : a.shape[1], :]


def _mask_rank(mask, n):
    m = mask.astype(jnp.int32)
    c = m
    sh = 1
    while sh < n:
        c = c + _shr_last(c, sh)
        sh *= 2
    return c - m


def _ball_kernel(xr, yr, zr, cxr, cyr, czr, o_ref, *, r2, ns, n, rb):
    b = pl.program_id(0)
    x = xr[0]
    y = yr[0]
    z = zr[0]
    cx = cxr[0, 0]
    cy = cyr[0, 0]
    cz = czr[0, 0]
    dx = cx - x
    dy = cy - y
    dz = cz - z
    d2 = dx * dx + dy * dy + dz * dz
    mask = d2 <= r2
    rank = jnp.where(mask, _mask_rank(mask, n), -1)
    kiota = lax.broadcasted_iota(jnp.int32, (rb, n), 1)
    cols = [jnp.min(jnp.where(rank == s, kiota, n), axis=1, keepdims=True)
            for s in range(ns)]
    idx = jnp.concatenate(cols, axis=1)
    first = idx[:, 0:1]
    idx = jnp.where(idx == n, first, idx)
    idx = jnp.where(idx == n, 0, idx)
    o_ref[0, 0] = idx + b * n


def _ball(xyz, new_xyz, radius, ns):
    b, n, _ = xyz.shape
    npoint = new_xyz.shape[1]
    rb = min(32, npoint)
    xr = xyz[..., 0].reshape(b, 1, n)
    yr = xyz[..., 1].reshape(b, 1, n)
    zr = xyz[..., 2].reshape(b, 1, n)
    cx = new_xyz[..., 0].reshape(b, npoint // rb, rb, 1)
    cy = new_xyz[..., 1].reshape(b, npoint // rb, rb, 1)
    cz = new_xyz[..., 2].reshape(b, npoint // rb, rb, 1)
    out = pl.pallas_call(
        functools.partial(_ball_kernel, r2=radius * radius, ns=ns, n=n,
                          rb=rb),
        grid=(b, npoint // rb),
        in_specs=[pl.BlockSpec((1, 1, n), lambda i, j: (i, 0, 0))] * 3
        + [pl.BlockSpec((1, 1, rb, 1), lambda i, j: (i, j, 0, 0))] * 3,
        out_specs=pl.BlockSpec((1, 1, rb, ns), lambda i, j: (i, j, 0, 0)),
        out_shape=jax.ShapeDtypeStruct((b, npoint // rb, rb, ns), jnp.int32),
    )(xr, yr, zr, cx, cy, cz)
    return out.reshape(b * npoint * ns)


def _sc_gather(table, idx):
    m = idx.shape[0]
    d = table.shape[1]
    sub = 128
    while 2 * sub * d * 4 > 229376:
        sub //= 2
    groups = m // sub
    per_w = groups // _NW
    idx2 = idx.reshape(groups, sub)
    mesh = plsc.VectorSubcoreMesh(core_axis_name="c", subcore_axis_name="s")

    @functools.partial(
        pl.kernel, mesh=mesh,
        out_type=jax.ShapeDtypeStruct((groups, sub, d), jnp.float32),
        scratch_types=[
            pltpu.VMEM((sub,), jnp.int32),
            pltpu.VMEM((sub,), jnp.int32),
            pltpu.VMEM((sub, d), jnp.float32),
            pltpu.VMEM((sub, d), jnp.float32),
            pltpu.SemaphoreType.DMA,
            pltpu.SemaphoreType.DMA,
        ],
    )
    def k(table_hbm, idx_hbm, out_hbm, i0, i1, r0, r1, s0, s1):
        wid = lax.axis_index("s") * _NC + lax.axis_index("c")
        base = wid * per_w
        pltpu.sync_copy(idx_hbm.at[base], i0)
        pltpu.async_copy(table_hbm.at[i0], r0, s0)

        def body(h, _):
            g0 = base + 2 * h
            pltpu.sync_copy(idx_hbm.at[g0 + 1], i1)
            pltpu.async_copy(table_hbm.at[i1], r1, s1)
            pltpu.make_async_copy(table_hbm.at[i0], r0, s0).wait()
            pltpu.sync_copy(r0, out_hbm.at[g0])

            @pl.when(2 * h + 2 < per_w)
            def _():
                pltpu.sync_copy(idx_hbm.at[g0 + 2], i0)
                pltpu.async_copy(table_hbm.at[i0], r0, s0)

            pltpu.make_async_copy(table_hbm.at[i1], r1, s1).wait()
            pltpu.sync_copy(r1, out_hbm.at[g0 + 1])
            return 0

        lax.fori_loop(0, per_w // 2, body, 0)

    return k(table, idx2).reshape(m, d)


def _sa_mlp_kernel(g_ref, c_ref, *refs, cb, ns, nlayer):
    o_ref = refs[3 * nlayer]
    h = (g_ref[...] - c_ref[...]).reshape(cb * ns, g_ref.shape[2])
    for l in range(nlayer):
        w, s, bt = refs[3 * l], refs[3 * l + 1], refs[3 * l + 2]
        h = jnp.dot(h, w[...], preferred_element_type=jnp.float32)
        h = jnp.maximum(h * s[...] + bt[...], 0.0)
    c_out = h.shape[1]
    o_ref[...] = jnp.max(h.reshape(cb, ns, c_out), axis=1)


def _sa_mlp(g, csub, layers, ns, cb):
    rows, d = csub.shape[0], csub.shape[2]
    gr = g.reshape(rows, ns, d)
    wargs = []
    wspecs = []
    for (wt, s, bt) in layers:
        c = wt.shape[1]
        wargs += [wt, s.reshape(1, c), bt.reshape(1, c)]
        wspecs += [
            pl.BlockSpec(wt.shape, lambda i: (0, 0)),
            pl.BlockSpec((1, c), lambda i: (0, 0)),
            pl.BlockSpec((1, c), lambda i: (0, 0)),
        ]
    c_out = layers[-1][0].shape[1]
    return pl.pallas_call(
        functools.partial(_sa_mlp_kernel, cb=cb, ns=ns, nlayer=len(layers)),
        grid=(rows // cb,),
        in_specs=[
            pl.BlockSpec((cb, ns, d), lambda i: (i, 0, 0)),
            pl.BlockSpec((cb, 1, d), lambda i: (i, 0, 0)),
        ] + wspecs,
        out_specs=pl.BlockSpec((cb, c_out), lambda i: (i, 0)),
        out_shape=jax.ShapeDtypeStruct((rows, c_out), jnp.float32),
    )(gr, csub, *wargs)


def _three_nn_kernel(xr, yr, zr, uxr, uyr, uzr, i_ref, w_ref, *, nk, rb):
    b = pl.program_id(0)
    x = xr[0]
    y = yr[0]
    z = zr[0]
    ux = uxr[0, 0]
    uy = uyr[0, 0]
    uz = uzr[0, 0]
    dx = ux - x
    dy = uy - y
    dz = uz - z
    d2 = dx * dx + dy * dy + dz * dz
    kiota = lax.broadcasted_iota(jnp.int32, (rb, nk), 1)
    vs, ids = [], []
    for _ in range(3):
        v = jnp.min(d2, axis=1, keepdims=True)
        ii = jnp.min(jnp.where(d2 == v, kiota, nk), axis=1, keepdims=True)
        d2 = jnp.where(kiota == ii, 1e30, d2)
        vs.append(v)
        ids.append(ii)
    r = [1.0 / (jnp.maximum(v, 0.0) + 1e-8) for v in vs]
    rs = r[0] + r[1] + r[2]
    zero = jnp.zeros((rb, 1), jnp.float32)
    w_ref[0, 0] = jnp.concatenate([r[0] / rs, r[1] / rs, r[2] / rs, zero],
                                  axis=1)
    izero = jnp.zeros((rb, 1), jnp.int32)
    i_ref[0, 0] = jnp.concatenate(ids + [izero], axis=1) + b * nk


def _three_nn(unknown, known):
    b, nu, _ = unknown.shape
    nk = known.shape[1]
    rb = min(64, nu)
    xr = known[..., 0].reshape(b, 1, nk)
    yr = known[..., 1].reshape(b, 1, nk)
    zr = known[..., 2].reshape(b, 1, nk)
    ux = unknown[..., 0].reshape(b, nu // rb, rb, 1)
    uy = unknown[..., 1].reshape(b, nu // rb, rb, 1)
    uz = unknown[..., 2].reshape(b, nu // rb, rb, 1)
    idx4, w4 = pl.pallas_call(
        functools.partial(_three_nn_kernel, nk=nk, rb=rb),
        grid=(b, nu // rb),
        in_specs=[pl.BlockSpec((1, 1, nk), lambda i, j: (i, 0, 0))] * 3
        + [pl.BlockSpec((1, 1, rb, 1), lambda i, j: (i, j, 0, 0))] * 3,
        out_specs=[
            pl.BlockSpec((1, 1, rb, 4), lambda i, j: (i, j, 0, 0)),
            pl.BlockSpec((1, 1, rb, 4), lambda i, j: (i, j, 0, 0)),
        ],
        out_shape=[
            jax.ShapeDtypeStruct((b, nu // rb, rb, 4), jnp.int32),
            jax.ShapeDtypeStruct((b, nu // rb, rb, 4), jnp.float32),
        ],
    )(xr, yr, zr, ux, uy, uz)
    return idx4.reshape(b * nu * 4), w4.reshape(b * nu, 4, 1)


def _fp_mlp_kernel(g_ref, w4_ref, s_ref, *refs, cb, nlayer, final):
    nw = 3 * nlayer + (2 if final else 0)
    o_ref = refs[nw]
    interp = jnp.sum(g_ref[...] * w4_ref[...], axis=1)
    h = jnp.concatenate([interp, s_ref[...]], axis=1)
    for l in range(nlayer):
        w, s, bt = refs[3 * l], refs[3 * l + 1], refs[3 * l + 2]
        h = jnp.dot(h, w[...], preferred_element_type=jnp.float32)
        h = jnp.maximum(h * s[...] + bt[...], 0.0)
    if final:
        wf, bf = refs[3 * nlayer], refs[3 * nlayer + 1]
        h = jnp.dot(h, wf[...], preferred_element_type=jnp.float32) + bf[...]
    o_ref[...] = h


def _fp_mlp(g4, w4, skip, layers, cb, final=None):
    rows = skip.shape[0]
    c = g4.shape[1]
    cs = skip.shape[1]
    gr = g4.reshape(rows, 4, c)
    wargs = []
    wspecs = []
    for (wt, s, bt) in layers:
        co = wt.shape[1]
        wargs += [wt, s.reshape(1, co), bt.reshape(1, co)]
        wspecs += [
            pl.BlockSpec(wt.shape, lambda i: (0, 0)),
            pl.BlockSpec((1, co), lambda i: (0, 0)),
            pl.BlockSpec((1, co), lambda i: (0, 0)),
        ]
    if final is not None:
        wf, bf = final
        co = wf.shape[1]
        wargs += [wf, bf.reshape(1, co)]
        wspecs += [
            pl.BlockSpec(wf.shape, lambda i: (0, 0)),
            pl.BlockSpec((1, co), lambda i: (0, 0)),
        ]
        c_out = co
    else:
        c_out = layers[-1][0].shape[1]
    return pl.pallas_call(
        functools.partial(_fp_mlp_kernel, cb=cb, nlayer=len(layers),
                          final=final is not None),
        grid=(rows // cb,),
        in_specs=[
            pl.BlockSpec((cb, 4, c), lambda i: (i, 0, 0)),
            pl.BlockSpec((cb, 4, 1), lambda i: (i, 0, 0)),
            pl.BlockSpec((cb, cs), lambda i: (i, 0)),
        ] + wspecs,
        out_specs=pl.BlockSpec((cb, c_out), lambda i: (i, 0)),
        out_shape=jax.ShapeDtypeStruct((rows, c_out), jnp.float32),
    )(gr, w4, skip, *wargs)


def _prep_layer(lp, cin_pad=None):
    w = lp["w"]
    if cin_pad is not None and cin_pad > w.shape[1]:
        w = jnp.pad(w, ((0, 0), (0, cin_pad - w.shape[1])))
    scale = lp["gamma"] / np.sqrt(1.0 + _BN_EPS)
    return w.T, scale, lp["beta"]


def _pad_rows(x, d):
    return jnp.pad(x, ((0, 0), (0, d - x.shape[1])))


def _sa_level(xyz, table, d, npoint, radii, nss, scale_params, cbs):
    b, n, _ = xyz.shape
    new_xyz = _fps(xyz, npoint)
    idx1 = _ball(xyz, new_xyz, radii[0], nss[0])
    idx2 = _ball(xyz, new_xyz, radii[1], nss[1])
    rows = _sc_gather(table, jnp.concatenate([idx1, idx2]))
    m1 = idx1.shape[0]
    csub = _pad_rows(new_xyz.reshape(b * npoint, 3), d)[:, None, :]
    outs = []
    for g, ns, lps, cb in ((rows[:m1], nss[0], scale_params[0], cbs),
                           (rows[m1:], nss[1], scale_params[1], cbs)):
        layers = [_prep_layer(lps[0], cin_pad=d)] + [_prep_layer(lp)
                                                     for lp in lps[1:]]
        outs.append(_sa_mlp(g, csub, layers, ns, cb))
    return new_xyz, jnp.concatenate(outs, axis=1)


def _fp_level(unknown, known, feat_known, skip, lps, cb, final=None):
    idx4, w4 = _three_nn(unknown, known)
    g4 = _sc_gather(feat_known, idx4)
    layers = [_prep_layer(lp) for lp in lps]
    return _fp_mlp(g4, w4, skip, layers, cb, final=final)


def kernel(pointcloud, params):
    b, n, _ = pointcloud.shape
    xyz0 = pointcloud[..., 0:3]
    feat0 = pointcloud.reshape(b * n, 9)[:, 3:]

    t1 = _pad_rows(pointcloud.reshape(b * n, 9), 128)
    new1, f1 = _sa_level(xyz0, t1, 128, 4096, (0.4, 0.8), (16, 32),
                         params["sa"][0], 64)
    t2 = _pad_rows(jnp.concatenate([new1.reshape(b * 4096, 3), f1], axis=1),
                   256)
    new2, f2 = _sa_level(new1, t2, 256, 1024, (0.8, 1.2), (16, 32),
                         params["sa"][1], 64)
    t3 = _pad_rows(jnp.concatenate([new2.reshape(b * 1024, 3), f2], axis=1),
                   640)
    new3, f3 = _sa_level(new2, t3, 640, 256, (1.2, 1.6), (16, 32),
                         params["sa"][2], 32)

    fp3 = _fp_level(new2, new3, f3, f2, params["fp"][2], 128)
    fp2 = _fp_level(new1, new2, fp3, f1, params["fp"][1], 256)
    fc = params["fc"]
    final = (fc["w2"].T, fc["b2"])
    lps = list(params["fp"][0]) + [fc["l1"]]
    out = _fp_level(xyz0, new1, fp2, feat0, lps, 512, final=final)
    return out.reshape(b, n, 13)
